# Initial kernel scaffold; baseline (speedup 1.0000x reference)
import jax
import jax.numpy as jnp
from jax.experimental import pallas as pl


def kernel(l1, xyz1, top_k, Wq, Wk):
    raise NotImplementedError("write your pallas kernel here")


# speedup vs baseline: 7.6385x; 7.6385x over previous
import functools

import jax
import jax.numpy as jnp
import numpy as np
from jax import lax
from jax.experimental import pallas as pl
from jax.experimental.pallas import tpu as pltpu
from jax.experimental.pallas import tpu_sc as plsc

B, CIN, COUT, N = 4, 128, 64, 4096
K = N // 4
NC, NS = 2, 16
NW = NC * NS
ROWS_W = (B * K) // NW


def _gather_body(l1t_hbm, xyz_hbm, idx_hbm, l1_out, xyz_out,
                 idx_v, idxg_v, xyz_tile, rows_v, xyz_rows, sem1, sem2):
    wid = lax.axis_index("s") * NC + lax.axis_index("c")
    base = wid * ROWS_W
    b = base // K
    pltpu.sync_copy(idx_hbm.at[pl.ds(base, ROWS_W)], idx_v)
    cp0 = pltpu.async_copy(xyz_hbm.at[pl.ds(b * (3 * N), 3 * N)], xyz_tile,
                           sem2)
    off = b * N
    for i in range(ROWS_W // 16):
        sl = pl.ds(i * 16, 16)
        idxg_v[sl] = idx_v[sl] + off
    cp1 = pltpu.async_copy(l1t_hbm.at[idxg_v], rows_v, sem1)
    cp0.wait()
    for d in range(3):
        doff = d * N
        for i in range(ROWS_W // 16):
            xyz_rows[pl.ds(d * ROWS_W + i * 16, 16)] = plsc.load_gather(
                xyz_tile, [idx_v[pl.ds(i * 16, 16)] + doff])
    cp1.wait()
    pltpu.sync_copy(rows_v, l1_out.at[pl.ds(base, ROWS_W)])
    for d in range(3):
        pltpu.sync_copy(xyz_rows.at[pl.ds(d * ROWS_W, ROWS_W)],
                        xyz_out.at[pl.ds(d * (B * K) + base, ROWS_W)])


@jax.jit
def _sc_gather(l1t, xyz_flat, idx_flat):
    mesh = plsc.VectorSubcoreMesh(core_axis_name="c", subcore_axis_name="s")
    f = pl.kernel(
        _gather_body,
        mesh=mesh,
        compiler_params=pltpu.CompilerParams(needs_layout_passes=False),
        out_type=(
            jax.ShapeDtypeStruct((B * K, CIN), jnp.float32),
            jax.ShapeDtypeStruct((3 * B * K,), jnp.float32),
        ),
        scratch_types=[
            pltpu.VMEM((ROWS_W,), jnp.int32),
            pltpu.VMEM((ROWS_W,), jnp.int32),
            pltpu.VMEM((3 * N,), jnp.float32),
            pltpu.VMEM((ROWS_W, CIN), jnp.float32),
            pltpu.VMEM((3 * ROWS_W,), jnp.float32),
            pltpu.SemaphoreType.DMA,
            pltpu.SemaphoreType.DMA,
        ],
    )
    return f(l1t, xyz_flat, idx_flat)


def kernel(l1, xyz1, top_k, Wq, Wk):
    q0 = jnp.einsum('oc,bc->bo', Wq, l1[:, :, 0])
    k1 = jnp.einsum('oc,bcn->bon', Wk, l1)
    energy = jnp.einsum('bc,bcm->bm', q0, k1)
    scale = np.sqrt(COUT)
    att = jax.nn.softmax(energy / scale, axis=-1)
    _, topk_idx = lax.top_k(att, K)
    topk_idx = topk_idx + (jnp.asarray(top_k, dtype=topk_idx.dtype) - K)

    idx_flat = topk_idx.reshape(B * K)
    l1t = jnp.transpose(l1, (0, 2, 1)).reshape(B * N, CIN)
    xyz_flat = xyz1.reshape(B * 3 * N)
    l1_out, xyz_out = _sc_gather(l1t, xyz_flat, idx_flat)
    p1 = jnp.transpose(xyz_out.reshape(3, B, K), (1, 2, 0))
    return (l1_out.reshape(B, K, CIN), p1)
